# baseline (speedup 1.0000x reference)
import dataclasses

import jax
import jax.numpy as jnp
from jax import lax
from jax.experimental import pallas as pl
from jax.experimental.pallas import tpu as pltpu
from jax.experimental.pallas import tpu_sc as plsc

N = 10000
E = 320000
D = 128
H = D // 2
NC = 2
NS = 16
NW = NC * NS
EPT = E // NW
EPS = E // NS
C = 50
NCH = EPS // C
RPS = 624

_mesh = plsc.VectorSubcoreMesh(
    core_axis_name="c", subcore_axis_name="s", num_cores=NC, num_subcores=NS
)

f32 = jnp.float32
i32 = jnp.int32

_sc_params = pltpu.CompilerParams(
    needs_layout_passes=False, use_tc_tiling_on_sc=False
)


def _bc16(v):
    return jnp.broadcast_to(v, (16,)).astype(i32)


def _deg_body(pkd_hbm, out_hbm, pkv, degv):
    c = lax.axis_index("c")
    s = lax.axis_index("s")
    w = s * NC + c
    pltpu.sync_copy(pkd_hbm.at[w], pkv)

    z16 = jnp.zeros((16,), i32)

    @pl.loop(0, N, step=16)
    def _(i):
        degv[0, pl.ds(i, 16)] = jnp.zeros((16,), f32)

    @pl.loop(0, EPT, step=16)
    def _(i):
        pk = pkv[0, pl.ds(i, 16)]
        idx = lax.shift_right_logical(pk, 16)
        val = (pk & 0xFFFF).astype(f32) * (1.0 / 65535.0)
        plsc.addupdate_scatter(degv, [z16, idx], val)

    pltpu.sync_copy(degv, out_hbm.at[w])


def _deg_partials(pkd):
    return pl.kernel(
        _deg_body,
        out_type=jax.ShapeDtypeStruct((NW, 1, N), f32),
        mesh=_mesh,
        scratch_types=[
            pltpu.VMEM((1, EPT), i32),
            pltpu.VMEM((1, N), f32),
        ],
        compiler_params=_sc_params,
    )(pkd)


def _agg_body(y_hbm, src_hbm, dst_hbm, ew_hbm, out_hbm,
              srcv, dstv, ewv, gb0, gb1, sb0, sb1, acc,
              gs0, gs1, ss0, ss1):
    c = lax.axis_index("c")
    s = lax.axis_index("s")
    pltpu.sync_copy(src_hbm.at[c, s], srcv)
    pltpu.sync_copy(dst_hbm.at[s], dstv)
    pltpu.sync_copy(ew_hbm.at[s], ewv)

    @pl.loop(0, C)
    def _(r):
        for k in range(H // 16):
            sb0[r, pl.ds(k * 16, 16)] = jnp.zeros((16,), f32)

    @pl.loop(0, RPS // 48)
    def _(k):
        pltpu.sync_copy(sb0.at[pl.ds(0, 48)],
                        acc.at[pl.ds(s * RPS + k * 48, 48)])

    @pl.when(s == NS - 1)
    def _():
        pltpu.sync_copy(sb0.at[pl.ds(0, N - NS * RPS)],
                        acc.at[pl.ds(NS * RPS, N - NS * RPS)])

    plsc.subcore_barrier()

    pltpu.async_copy(y_hbm.at[srcv.at[0, 0]], gb0, gs0)
    pltpu.async_copy(y_hbm.at[srcv.at[1, 0]], gb1, gs1)

    def do_chunk(cc, gb, sb, gs, ss):
        pltpu.make_async_copy(y_hbm.at[srcv.at[0, 0]], gb, gs).wait()

        @pl.when(cc >= 2)
        def _():
            pltpu.make_async_copy(sb, acc.at[dstv.at[0, 0]], ss).wait()

        @pl.loop(0, C, unroll=4)
        def _(e):
            ewb = plsc.load_gather(ewv, [_bc16(cc), _bc16(0), _bc16(e)])
            for k in range(H // 16):
                sl = pl.ds(k * 16, 16)
                sb[e, sl] = gb[e, sl] * ewb

        pltpu.async_copy(sb, acc.at[dstv.at[cc, 0]], ss, add=True)

        @pl.when(cc + 2 < NCH)
        def _():
            pltpu.async_copy(y_hbm.at[srcv.at[cc + 2, 0]], gb, gs)

    @pl.loop(0, NCH, step=2)
    def _(g):
        do_chunk(g, gb0, sb0, gs0, ss0)
        do_chunk(g + 1, gb1, sb1, gs1, ss1)

    pltpu.make_async_copy(sb0, acc.at[dstv.at[0, 0]], ss0).wait()
    pltpu.make_async_copy(sb1, acc.at[dstv.at[0, 0]], ss1).wait()
    plsc.subcore_barrier()
    pltpu.sync_copy(acc.at[pl.ds(s * RPS, RPS)],
                    out_hbm.at[c, pl.ds(s * RPS, RPS)])

    @pl.when(s == NS - 1)
    def _():
        pltpu.sync_copy(acc.at[pl.ds(NS * RPS, N - NS * RPS)],
                        out_hbm.at[c, pl.ds(NS * RPS, N - NS * RPS)])


def _aggregate(y_pair, src5, dst4, ew4):
    y_flat = y_pair.reshape(NC * N, H)
    return pl.kernel(
        _agg_body,
        out_type=jax.ShapeDtypeStruct((NC, N, H), f32),
        mesh=_mesh,
        scratch_types=[
            pltpu.VMEM((NCH, 1, C), i32),
            pltpu.VMEM((NCH, 1, C), i32),
            pltpu.VMEM((NCH, 1, C), f32),
            pltpu.VMEM((C, H), f32),
            pltpu.VMEM((C, H), f32),
            pltpu.VMEM((C, H), f32),
            pltpu.VMEM((C, H), f32),
            pltpu.VMEM_SHARED((N, H), f32),
            pltpu.SemaphoreType.DMA,
            pltpu.SemaphoreType.DMA,
            pltpu.SemaphoreType.DMA,
            pltpu.SemaphoreType.DMA,
        ],
        compiler_params=_sc_params,
    )(y_flat, src5, dst4, ew4)


_PREC = lax.Precision.HIGHEST


def _split(y):
    return jnp.stack([y[:, :H], y[:, H:]], axis=0)


def _tc1_body(degp_ref, x_ref, w_ref, y_ref, dis_ref):
    deg = lax.dot_general(degp_ref[...], jnp.ones((NW, 1), f32),
                          (((0,), (0,)), ((), ())), precision=_PREC)
    dis = lax.rsqrt(deg + 1.0)
    xw = jnp.dot(x_ref[...], w_ref[...], preferred_element_type=f32,
                 precision=_PREC)
    y_ref[...] = _split(xw * dis)
    dis_ref[...] = dis


def _tc1(degp, x, W1):
    return pl.pallas_call(
        _tc1_body,
        out_shape=[
            jax.ShapeDtypeStruct((NC, N, H), f32),
            jax.ShapeDtypeStruct((N, 1), f32),
        ],
    )(degp, x, W1)


def _tc_mid_body(accp_ref, y_ref, dis_ref, b_ref, w_ref, y2_ref, h_ref):
    m = accp_ref[...] + y_ref[...]
    a = jnp.concatenate([m[0], m[1]], axis=-1)
    dis = dis_ref[...]
    h = jnp.maximum(a * dis + b_ref[...], 0.0)
    y2_ref[...] = _split(jnp.dot(h, w_ref[...], preferred_element_type=f32,
                                 precision=_PREC) * dis)
    h_ref[...] = h


_B = 2000


def _tc_mid(accp, y, dis, b, W):
    return pl.pallas_call(
        _tc_mid_body,
        grid=(N // _B,),
        in_specs=[
            pl.BlockSpec((NC, _B, H), lambda i: (0, i, 0)),
            pl.BlockSpec((NC, _B, H), lambda i: (0, i, 0)),
            pl.BlockSpec((_B, 1), lambda i: (i, 0)),
            pl.BlockSpec((D,), lambda i: (0,)),
            pl.BlockSpec((D, D), lambda i: (0, 0)),
        ],
        out_specs=[
            pl.BlockSpec((NC, _B, H), lambda i: (0, i, 0)),
            pl.BlockSpec((_B, D), lambda i: (i, 0)),
        ],
        out_shape=[
            jax.ShapeDtypeStruct((NC, N, H), f32),
            jax.ShapeDtypeStruct((N, D), f32),
        ],
    )(accp, y, dis, b, W)


def _tc_out_body(h_ref, wfc_ref, bfc_ref, out_ref):
    out_ref[...] = jnp.dot(h_ref[...], wfc_ref[...], preferred_element_type=f32,
                           precision=_PREC) + bfc_ref[...]


def _tc_out(h, Wfc, bfc):
    return pl.pallas_call(
        _tc_out_body,
        out_shape=jax.ShapeDtypeStruct((N, D), f32),
    )(h, Wfc, bfc)


def kernel(x, edge_index, edge_attr, W1, b1, W2, b2, Wfc, bfc):
    src = edge_index[0].astype(i32)
    dst = edge_index[1].astype(i32)
    ew = edge_attr.reshape(E).astype(f32)
    src5 = jnp.stack([src, src + N]).reshape(NC, NS, NCH, 1, C)
    dst4 = dst.reshape(NS, NCH, 1, C)
    ew4 = ew.reshape(NS, NCH, 1, C)
    ew_q = jnp.round(ew * 65535.0).astype(i32)
    pkd = (ew_q | (dst << 16)).reshape(NW, 1, EPT)

    degp = _deg_partials(pkd).reshape(NW, N)
    y1, dis = _tc1(degp, x, W1)

    def _layer(carry, bw):
        y, _ = carry
        b_l, W_l = bw
        accp = _aggregate(y, src5, dst4, ew4)
        y_next, h = _tc_mid(accp, y, dis, b_l, W_l)
        return (y_next, h), None

    bs = jnp.stack([b1, b2])
    Ws = jnp.stack([W2, Wfc])
    (_, h2), _ = lax.scan(_layer, (y1, jnp.zeros((N, D), f32)), (bs, Ws))
    return _tc_out(h2, Wfc, bfc)


# speedup vs baseline: 1.1315x; 1.1315x over previous
import dataclasses

import jax
import jax.numpy as jnp
from jax import lax
from jax.experimental import pallas as pl
from jax.experimental.pallas import tpu as pltpu
from jax.experimental.pallas import tpu_sc as plsc

N = 10000
E = 320000
D = 128
H = D // 2
NC = 2
NS = 16
NW = NC * NS
EPT = E // NW
EPS = E // NS
C = 100
NCH = EPS // C
RPS = 624

_mesh = plsc.VectorSubcoreMesh(
    core_axis_name="c", subcore_axis_name="s", num_cores=NC, num_subcores=NS
)

f32 = jnp.float32
i32 = jnp.int32

_sc_params = pltpu.CompilerParams(
    needs_layout_passes=False, use_tc_tiling_on_sc=False
)


def _bc16(v):
    return jnp.broadcast_to(v, (16,)).astype(i32)


def _deg_body(pkd_hbm, out_hbm, pkv, degv):
    c = lax.axis_index("c")
    s = lax.axis_index("s")
    w = s * NC + c
    pltpu.sync_copy(pkd_hbm.at[w], pkv)

    z16 = jnp.zeros((16,), i32)

    @pl.loop(0, N, step=16)
    def _(i):
        degv[0, pl.ds(i, 16)] = jnp.zeros((16,), f32)

    @pl.loop(0, EPT, step=16)
    def _(i):
        pk = pkv[0, pl.ds(i, 16)]
        idx = lax.shift_right_logical(pk, 16)
        val = (pk & 0xFFFF).astype(f32) * (1.0 / 65535.0)
        plsc.addupdate_scatter(degv, [z16, idx], val)

    pltpu.sync_copy(degv, out_hbm.at[w])


def _deg_partials(pkd):
    return pl.kernel(
        _deg_body,
        out_type=jax.ShapeDtypeStruct((NW, 1, N), f32),
        mesh=_mesh,
        scratch_types=[
            pltpu.VMEM((1, EPT), i32),
            pltpu.VMEM((1, N), f32),
        ],
        compiler_params=_sc_params,
    )(pkd)


def _agg_body(y_hbm, src_hbm, dst_hbm, ew_hbm, out_hbm,
              srcv, dstv, ewv, gb0, gb1, sb0, sb1, acc,
              gs0, gs1, ss0, ss1):
    c = lax.axis_index("c")
    s = lax.axis_index("s")
    pltpu.sync_copy(src_hbm.at[c, s], srcv)
    pltpu.sync_copy(dst_hbm.at[s], dstv)
    pltpu.sync_copy(ew_hbm.at[s], ewv)

    @pl.loop(0, C)
    def _(r):
        for k in range(H // 16):
            sb0[r, pl.ds(k * 16, 16)] = jnp.zeros((16,), f32)

    @pl.loop(0, RPS // 48)
    def _(k):
        pltpu.sync_copy(sb0.at[pl.ds(0, 48)],
                        acc.at[pl.ds(s * RPS + k * 48, 48)])

    @pl.when(s == NS - 1)
    def _():
        pltpu.sync_copy(sb0.at[pl.ds(0, N - NS * RPS)],
                        acc.at[pl.ds(NS * RPS, N - NS * RPS)])

    plsc.subcore_barrier()

    pltpu.async_copy(y_hbm.at[srcv.at[0, 0]], gb0, gs0)
    pltpu.async_copy(y_hbm.at[srcv.at[1, 0]], gb1, gs1)

    def do_chunk(cc, gb, sb, gs, ss):
        pltpu.make_async_copy(y_hbm.at[srcv.at[0, 0]], gb, gs).wait()

        @pl.when(cc >= 2)
        def _():
            pltpu.make_async_copy(sb, acc.at[dstv.at[0, 0]], ss).wait()

        @pl.loop(0, C, unroll=4)
        def _(e):
            ewb = plsc.load_gather(ewv, [_bc16(cc), _bc16(0), _bc16(e)])
            for k in range(H // 16):
                sl = pl.ds(k * 16, 16)
                sb[e, sl] = gb[e, sl] * ewb

        pltpu.async_copy(sb, acc.at[dstv.at[cc, 0]], ss, add=True)

        @pl.when(cc + 2 < NCH)
        def _():
            pltpu.async_copy(y_hbm.at[srcv.at[cc + 2, 0]], gb, gs)

    @pl.loop(0, NCH, step=2)
    def _(g):
        do_chunk(g, gb0, sb0, gs0, ss0)
        do_chunk(g + 1, gb1, sb1, gs1, ss1)

    pltpu.make_async_copy(sb0, acc.at[dstv.at[0, 0]], ss0).wait()
    pltpu.make_async_copy(sb1, acc.at[dstv.at[0, 0]], ss1).wait()
    plsc.subcore_barrier()
    pltpu.sync_copy(acc.at[pl.ds(s * RPS, RPS)],
                    out_hbm.at[c, pl.ds(s * RPS, RPS)])

    @pl.when(s == NS - 1)
    def _():
        pltpu.sync_copy(acc.at[pl.ds(NS * RPS, N - NS * RPS)],
                        out_hbm.at[c, pl.ds(NS * RPS, N - NS * RPS)])


def _aggregate(y_pair, src5, dst4, ew4):
    y_flat = y_pair.reshape(NC * N, H)
    return pl.kernel(
        _agg_body,
        out_type=jax.ShapeDtypeStruct((NC, N, H), f32),
        mesh=_mesh,
        scratch_types=[
            pltpu.VMEM((NCH, 1, C), i32),
            pltpu.VMEM((NCH, 1, C), i32),
            pltpu.VMEM((NCH, 1, C), f32),
            pltpu.VMEM((C, H), f32),
            pltpu.VMEM((C, H), f32),
            pltpu.VMEM((C, H), f32),
            pltpu.VMEM((C, H), f32),
            pltpu.VMEM_SHARED((N, H), f32),
            pltpu.SemaphoreType.DMA,
            pltpu.SemaphoreType.DMA,
            pltpu.SemaphoreType.DMA,
            pltpu.SemaphoreType.DMA,
        ],
        compiler_params=_sc_params,
    )(y_flat, src5, dst4, ew4)


_PREC = lax.Precision.HIGHEST


def _split(y):
    return jnp.stack([y[:, :H], y[:, H:]], axis=0)


def _tc1_body(degp_ref, x_ref, w_ref, y_ref, dis_ref):
    deg = lax.dot_general(degp_ref[...], jnp.ones((NW, 1), f32),
                          (((0,), (0,)), ((), ())), precision=_PREC)
    dis = lax.rsqrt(deg + 1.0)
    xw = jnp.dot(x_ref[...], w_ref[...], preferred_element_type=f32,
                 precision=_PREC)
    y_ref[...] = _split(xw * dis)
    dis_ref[...] = dis


def _tc1(degp, x, W1):
    return pl.pallas_call(
        _tc1_body,
        out_shape=[
            jax.ShapeDtypeStruct((NC, N, H), f32),
            jax.ShapeDtypeStruct((N, 1), f32),
        ],
    )(degp, x, W1)


def _tc_mid_body(accp_ref, y_ref, dis_ref, b_ref, w_ref, y2_ref, h_ref):
    m = accp_ref[...] + y_ref[...]
    a = jnp.concatenate([m[0], m[1]], axis=-1)
    dis = dis_ref[...]
    h = jnp.maximum(a * dis + b_ref[...], 0.0)
    y2_ref[...] = _split(jnp.dot(h, w_ref[...], preferred_element_type=f32,
                                 precision=_PREC) * dis)
    h_ref[...] = h


_B = 2000


def _tc_mid(accp, y, dis, b, W):
    return pl.pallas_call(
        _tc_mid_body,
        grid=(N // _B,),
        in_specs=[
            pl.BlockSpec((NC, _B, H), lambda i: (0, i, 0)),
            pl.BlockSpec((NC, _B, H), lambda i: (0, i, 0)),
            pl.BlockSpec((_B, 1), lambda i: (i, 0)),
            pl.BlockSpec((D,), lambda i: (0,)),
            pl.BlockSpec((D, D), lambda i: (0, 0)),
        ],
        out_specs=[
            pl.BlockSpec((NC, _B, H), lambda i: (0, i, 0)),
            pl.BlockSpec((_B, D), lambda i: (i, 0)),
        ],
        out_shape=[
            jax.ShapeDtypeStruct((NC, N, H), f32),
            jax.ShapeDtypeStruct((N, D), f32),
        ],
    )(accp, y, dis, b, W)


def _tc_out_body(h_ref, wfc_ref, bfc_ref, out_ref):
    out_ref[...] = jnp.dot(h_ref[...], wfc_ref[...], preferred_element_type=f32,
                           precision=_PREC) + bfc_ref[...]


def _tc_out(h, Wfc, bfc):
    return pl.pallas_call(
        _tc_out_body,
        out_shape=jax.ShapeDtypeStruct((N, D), f32),
    )(h, Wfc, bfc)


def kernel(x, edge_index, edge_attr, W1, b1, W2, b2, Wfc, bfc):
    src = edge_index[0].astype(i32)
    dst = edge_index[1].astype(i32)
    ew = edge_attr.reshape(E).astype(f32)
    src5 = jnp.stack([src, src + N]).reshape(NC, NS, NCH, 1, C)
    dst4 = dst.reshape(NS, NCH, 1, C)
    ew4 = ew.reshape(NS, NCH, 1, C)
    ew_q = jnp.round(ew * 65535.0).astype(i32)
    pkd = (ew_q | (dst << 16)).reshape(NW, 1, EPT)

    degp = _deg_partials(pkd).reshape(NW, N)
    y1, dis = _tc1(degp, x, W1)

    def _layer(carry, bw):
        y, _ = carry
        b_l, W_l = bw
        accp = _aggregate(y, src5, dst4, ew4)
        y_next, h = _tc_mid(accp, y, dis, b_l, W_l)
        return (y_next, h), None

    bs = jnp.stack([b1, b2])
    Ws = jnp.stack([W2, Wfc])
    (_, h2), _ = lax.scan(_layer, (y1, jnp.zeros((N, D), f32)), (bs, Ws))
    return _tc_out(h2, Wfc, bfc)
